# baseline (speedup 1.0000x reference)
import functools

import jax
import jax.numpy as jnp
from jax import lax
from jax.experimental import pallas as pl
from jax.experimental.pallas import tpu as pltpu
from jax.experimental.pallas import tpu_sc as plsc

_L = 16
_NW = 32


def kernel(x, img_idx, white_balance_parameter):
    n = x.shape[0]
    n_rows = white_balance_parameter.shape[0]
    x_t = x.T
    idx_1d = img_idx.astype(jnp.int32).reshape(n)
    tbl_cm = white_balance_parameter.T.reshape(3 * n_rows)

    per_w = n // _NW
    chunk = 4096
    n_chunks = per_w // chunk

    mesh = plsc.VectorSubcoreMesh(core_axis_name="c", subcore_axis_name="s")

    @functools.partial(
        pl.kernel,
        mesh=mesh,
        compiler_params=pltpu.CompilerParams(needs_layout_passes=False),
        out_type=jax.ShapeDtypeStruct((3, n), jnp.float32),
        scratch_types=[
            pltpu.VMEM((n_rows,), jnp.float32),
            pltpu.VMEM((n_rows,), jnp.float32),
            pltpu.VMEM((n_rows,), jnp.float32),
            pltpu.VMEM((chunk,), jnp.int32),
            pltpu.VMEM((chunk,), jnp.int32),
            pltpu.VMEM((8, chunk), jnp.float32),
            pltpu.VMEM((8, chunk), jnp.float32),
            pltpu.SemaphoreType.DMA,
            pltpu.SemaphoreType.DMA,
            pltpu.SemaphoreType.DMA,
            pltpu.SemaphoreType.DMA,
        ],
    )
    def wb(
        x_hbm, idx_hbm, tbl_hbm, out_hbm,
        t0_v, t1_v, t2_v, idx0_v, idx1_v, b0_v, b1_v,
        sem_in0, sem_in1, sem_out0, sem_out1,
    ):
        wid = lax.axis_index("s") * 2 + lax.axis_index("c")
        ray0 = wid * per_w
        tbls = (t0_v, t1_v, t2_v)
        idx_bufs = (idx0_v, idx1_v)
        bufs = (b0_v, b1_v)
        sems_in = (sem_in0, sem_in1)
        sems_out = (sem_out0, sem_out1)

        def in_copies(p, cidx):
            rbase = ray0 + cidx * chunk
            return [
                pltpu.make_async_copy(
                    idx_hbm.at[pl.ds(rbase, chunk)], idx_bufs[p], sems_in[p]
                ),
                pltpu.make_async_copy(
                    x_hbm.at[pl.ds(0, 3), pl.ds(rbase, chunk)],
                    bufs[p].at[pl.ds(0, 3)],
                    sems_in[p],
                ),
            ]

        def out_copies(p, cidx):
            rbase = ray0 + cidx * chunk
            return [
                pltpu.make_async_copy(
                    bufs[p].at[pl.ds(4, 3)],
                    out_hbm.at[pl.ds(0, 3), pl.ds(rbase, chunk)],
                    sems_out[p],
                )
            ]

        for cp in in_copies(0, 0):
            cp.start()
        tbl_copies = [
            pltpu.make_async_copy(
                tbl_hbm.at[pl.ds(c * n_rows, n_rows)], tbls[c], sem_out0
            )
            for c in range(3)
        ]
        for cp in tbl_copies:
            cp.start()
        for cp in tbl_copies:
            cp.wait()

        for cidx in range(n_chunks):
            p = cidx % 2
            if cidx + 1 < n_chunks:
                for cp in in_copies(1 - p, cidx + 1):
                    cp.start()

            for cp in in_copies(p, cidx):
                cp.wait()

            if cidx >= 2:
                for cp in out_copies(p, cidx - 2):
                    cp.wait()

            @plsc.parallel_loop(0, chunk // _L, unroll=8)
            def _(g, p=p):
                idxv = idx_bufs[p][pl.ds(_L * g, _L)]
                xs = [bufs[p][c, pl.ds(_L * g, _L)] for c in range(3)]
                tvs = [plsc.load_gather(tbls[c], [idxv]) for c in range(3)]
                for c in range(3):
                    bufs[p][4 + c, pl.ds(_L * g, _L)] = xs[c] * tvs[c]

            for cp in out_copies(p, cidx):
                cp.start()

        for p, cidx in ((0, n_chunks - 2), (1, n_chunks - 1)):
            for cp in out_copies(p, cidx):
                cp.wait()

    out_t = wb(x_t, idx_1d, tbl_cm)
    return out_t.T


# speedup vs baseline: 1.1019x; 1.1019x over previous
import functools

import jax
import jax.numpy as jnp
from jax import lax
from jax.experimental import pallas as pl
from jax.experimental.pallas import tpu as pltpu
from jax.experimental.pallas import tpu_sc as plsc

_L = 16
_NW = 32


def kernel(x, img_idx, white_balance_parameter):
    n = x.shape[0]
    n_rows = white_balance_parameter.shape[0]
    x_t = x.T
    idx_1d = img_idx.astype(jnp.int32).reshape(n)
    tbl_cm = white_balance_parameter.T.reshape(3 * n_rows)

    per_w = n // _NW
    chunk = 4096
    n_chunks = per_w // chunk

    mesh = plsc.VectorSubcoreMesh(core_axis_name="c", subcore_axis_name="s")

    @functools.partial(
        pl.kernel,
        mesh=mesh,
        compiler_params=pltpu.CompilerParams(needs_layout_passes=False),
        out_type=jax.ShapeDtypeStruct((3, n), jnp.float32),
        scratch_types=[
            pltpu.VMEM((n_rows,), jnp.float32),
            pltpu.VMEM((n_rows,), jnp.float32),
            pltpu.VMEM((n_rows,), jnp.float32),
            pltpu.VMEM((chunk,), jnp.int32),
            pltpu.VMEM((chunk,), jnp.int32),
            pltpu.VMEM((8, chunk), jnp.float32),
            pltpu.VMEM((8, chunk), jnp.float32),
            pltpu.SemaphoreType.DMA,
            pltpu.SemaphoreType.DMA,
            pltpu.SemaphoreType.DMA,
            pltpu.SemaphoreType.DMA,
        ],
    )
    def wb(
        x_hbm, idx_hbm, tbl_hbm, out_hbm,
        t0_v, t1_v, t2_v, idx0_v, idx1_v, b0_v, b1_v,
        sem_in0, sem_in1, sem_out0, sem_out1,
    ):
        wid = lax.axis_index("s") * 2 + lax.axis_index("c")
        ray0 = wid * per_w
        tbls = (t0_v, t1_v, t2_v)
        idx_bufs = (idx0_v, idx1_v)
        bufs = (b0_v, b1_v)
        sems_in = (sem_in0, sem_in1)
        sems_out = (sem_out0, sem_out1)

        def in_copies(p, cidx):
            rbase = ray0 + cidx * chunk
            return [
                pltpu.make_async_copy(
                    idx_hbm.at[pl.ds(rbase, chunk)], idx_bufs[p], sems_in[p]
                ),
                pltpu.make_async_copy(
                    x_hbm.at[pl.ds(0, 3), pl.ds(rbase, chunk)],
                    bufs[p].at[pl.ds(0, 3)],
                    sems_in[p],
                ),
            ]

        def out_copies(p, cidx):
            rbase = ray0 + cidx * chunk
            return [
                pltpu.make_async_copy(
                    bufs[p].at[pl.ds(4, 3)],
                    out_hbm.at[pl.ds(0, 3), pl.ds(rbase, chunk)],
                    sems_out[p],
                )
            ]

        for cp in in_copies(0, 0):
            cp.start()
        tbl_copies = [
            pltpu.make_async_copy(
                tbl_hbm.at[pl.ds(c * n_rows, n_rows)], tbls[c], sem_out0
            )
            for c in range(3)
        ]
        for cp in tbl_copies:
            cp.start()
        for cp in tbl_copies:
            cp.wait()

        def do_chunk(p, cidx):
            @pl.when(cidx + 1 < n_chunks)
            def _():
                for cp in in_copies(1 - p, cidx + 1):
                    cp.start()

            for cp in in_copies(p, cidx):
                cp.wait()

            @pl.when(cidx >= 2)
            def _():
                for cp in out_copies(p, cidx - 2):
                    cp.wait()

            @plsc.parallel_loop(0, chunk // _L, unroll=8)
            def _(g):
                idxv = idx_bufs[p][pl.ds(_L * g, _L)]
                xs = [bufs[p][c, pl.ds(_L * g, _L)] for c in range(3)]
                tvs = [plsc.load_gather(tbls[c], [idxv]) for c in range(3)]
                for c in range(3):
                    bufs[p][4 + c, pl.ds(_L * g, _L)] = xs[c] * tvs[c]

            for cp in out_copies(p, cidx):
                cp.start()

        def loop_body(base_cidx, carry):
            do_chunk(0, base_cidx)
            do_chunk(1, base_cidx + 1)
            return carry

        lax.fori_loop(0, n_chunks // 2, lambda i, c: loop_body(2 * i, c), 0)

        for p, cidx in ((0, n_chunks - 2), (1, n_chunks - 1)):
            for cp in out_copies(p, cidx):
                cp.wait()

    out_t = wb(x_t, idx_1d, tbl_cm)
    return out_t.T


# speedup vs baseline: 1.1065x; 1.0042x over previous
import functools

import jax
import jax.numpy as jnp
from jax import lax
from jax.experimental import pallas as pl
from jax.experimental.pallas import tpu as pltpu
from jax.experimental.pallas import tpu_sc as plsc

_L = 16
_NW = 32


def kernel(x, img_idx, white_balance_parameter):
    n = x.shape[0]
    n_rows = white_balance_parameter.shape[0]
    x_t = x.T
    idx_1d = img_idx.astype(jnp.int32).reshape(n)
    tbl_cm = white_balance_parameter.T.reshape(3 * n_rows)

    per_w = n // _NW
    chunk = 4096
    n_chunks = per_w // chunk

    mesh = plsc.VectorSubcoreMesh(core_axis_name="c", subcore_axis_name="s")

    @functools.partial(
        pl.kernel,
        mesh=mesh,
        compiler_params=pltpu.CompilerParams(needs_layout_passes=False),
        out_type=jax.ShapeDtypeStruct((3, n), jnp.float32),
        scratch_types=[
            pltpu.VMEM((n_rows,), jnp.float32),
            pltpu.VMEM((n_rows,), jnp.float32),
            pltpu.VMEM((n_rows,), jnp.float32),
            pltpu.VMEM((chunk,), jnp.int32),
            pltpu.VMEM((chunk,), jnp.int32),
            pltpu.VMEM((8, chunk), jnp.float32),
            pltpu.VMEM((8, chunk), jnp.float32),
            pltpu.SemaphoreType.DMA,
            pltpu.SemaphoreType.DMA,
            pltpu.SemaphoreType.DMA,
            pltpu.SemaphoreType.DMA,
        ],
    )
    def wb(
        x_hbm, idx_hbm, tbl_hbm, out_hbm,
        t0_v, t1_v, t2_v, idx0_v, idx1_v, b0_v, b1_v,
        sem_in0, sem_in1, sem_out0, sem_out1,
    ):
        wid = lax.axis_index("s") * 2 + lax.axis_index("c")
        ray0 = wid * per_w
        tbls = (t0_v, t1_v, t2_v)
        idx_bufs = (idx0_v, idx1_v)
        bufs = (b0_v, b1_v)
        sems_in = (sem_in0, sem_in1)
        sems_out = (sem_out0, sem_out1)

        def in_copies(p, cidx):
            rbase = ray0 + cidx * chunk
            return [
                pltpu.make_async_copy(
                    idx_hbm.at[pl.ds(rbase, chunk)], idx_bufs[p], sems_in[p]
                ),
                pltpu.make_async_copy(
                    x_hbm.at[pl.ds(0, 3), pl.ds(rbase, chunk)],
                    bufs[p].at[pl.ds(0, 3)],
                    sems_in[p],
                ),
            ]

        def out_copies(p, cidx):
            rbase = ray0 + cidx * chunk
            return [
                pltpu.make_async_copy(
                    bufs[p].at[pl.ds(4, 3)],
                    out_hbm.at[pl.ds(0, 3), pl.ds(rbase, chunk)],
                    sems_out[p],
                )
            ]

        for cp in in_copies(0, 0):
            cp.start()
        tbl_copies = [
            pltpu.make_async_copy(
                tbl_hbm.at[pl.ds(c * n_rows, n_rows)], tbls[c], sem_out0
            )
            for c in range(3)
        ]
        for cp in tbl_copies:
            cp.start()
        for cp in tbl_copies:
            cp.wait()

        def do_chunk(p, cidx):
            @pl.when(cidx + 1 < n_chunks)
            def _():
                for cp in in_copies(1 - p, cidx + 1):
                    cp.start()

            @pl.when(cidx >= 2)
            def _():
                for cp in out_copies(p, cidx - 2):
                    cp.wait()

            for cp in in_copies(p, cidx):
                cp.wait()

            @plsc.parallel_loop(0, chunk // _L, unroll=8)
            def _(g):
                idxv = idx_bufs[p][pl.ds(_L * g, _L)]
                xs = [bufs[p][c, pl.ds(_L * g, _L)] for c in range(3)]
                tvs = [plsc.load_gather(tbls[c], [idxv]) for c in range(3)]
                for c in range(3):
                    bufs[p][4 + c, pl.ds(_L * g, _L)] = xs[c] * tvs[c]

            for cp in out_copies(p, cidx):
                cp.start()

        def loop_body(base_cidx, carry):
            do_chunk(0, base_cidx)
            do_chunk(1, base_cidx + 1)
            return carry

        lax.fori_loop(0, n_chunks // 2, lambda i, c: loop_body(2 * i, c), 0)

        for p, cidx in ((0, n_chunks - 2), (1, n_chunks - 1)):
            for cp in out_copies(p, cidx):
                cp.wait()

    out_t = wb(x_t, idx_1d, tbl_cm)
    return out_t.T
